# baseline (speedup 1.0000x reference)
import jax
import jax.numpy as jnp
from jax.experimental import pallas as pl
from jax.experimental.pallas import tpu as pltpu

B, N, PDIM = 8, 2048, 3
KNN = 5
CH = 64
RB = 256
NB = N // RB
CNT = float(B * N * KNN)


def _knn_kernel(p_ref, pt_ref, prow_ref, wt_ref, umax_ref, umin_ref, stats_ref):
    pts = p_ref[0]
    ptsT = pt_ref[0]
    rows = prow_ref[0]
    wT = wt_ref[...]

    sq_all = jnp.sum(ptsT * ptsT, axis=0, keepdims=True)
    sq_row = jnp.sum(rows * rows, axis=1, keepdims=True)
    rbf = rows.astype(jnp.bfloat16).astype(jnp.float32)
    tbf = ptsT.astype(jnp.bfloat16).astype(jnp.float32)
    cross = (rbf[:, 0:1] * tbf[0:1, :]
             + rbf[:, 1:2] * tbf[1:2, :]
             + rbf[:, 2:3] * tbf[2:3, :])
    d2 = sq_row + sq_all - 2.0 * cross

    iota = jax.lax.broadcasted_iota(jnp.int32, (RB, N), 1)
    umax = None
    umin = None
    m1 = jnp.zeros((1, PDIM), jnp.float32)
    m2 = jnp.zeros((PDIM, PDIM), jnp.float32)
    for r in range(KNN):
        mval = jnp.min(d2, axis=1, keepdims=True)
        idx = jnp.min(jnp.where(d2 <= mval, iota, N),
                      axis=1, keepdims=True)
        mask = iota == idx
        maskf = mask.astype(jnp.float32)
        nbr = jax.lax.dot_general(maskf, pts, (((1,), (0,)), ((), ())),
                                  precision=jax.lax.Precision.HIGHEST,
                                  preferred_element_type=jnp.float32)
        e = nbr - rows
        ebf = e.astype(jnp.bfloat16).astype(jnp.float32)
        z = (ebf[:, 0:1] * wT[0:1, :]
             + ebf[:, 1:2] * wT[1:2, :]
             + ebf[:, 2:3] * wT[2:3, :])
        umax = z if r == 0 else jnp.maximum(umax, z)
        umin = z if r == 0 else jnp.minimum(umin, z)
        m1 = m1 + jnp.sum(e, axis=0, keepdims=True)
        m2 = m2 + jax.lax.dot_general(e, e, (((0,), (0,)), ((), ())),
                                      preferred_element_type=jnp.float32)
        d2 = jnp.where(mask, jnp.inf, d2)

    umax_ref[0] = umax
    umin_ref[0] = umin
    blk = jnp.concatenate(
        [jnp.concatenate([m1, jnp.zeros((1, 128 - PDIM), jnp.float32)], axis=1),
         jnp.concatenate([m2, jnp.zeros((PDIM, 128 - PDIM), jnp.float32)], axis=1),
         jnp.zeros((4, 128), jnp.float32)], axis=0)
    first = pl.program_id(1) == 0

    @pl.when(first)
    def _():
        stats_ref[0] = blk

    @pl.when(jnp.logical_not(first))
    def _():
        stats_ref[0] = stats_ref[0] + blk


def _head_kernel(umax_ref, umin_ref, stats_ref, wt_ref, g_ref, bta_ref,
                 w1_ref, b1_ref, w2_ref, b2_ref, w3_ref, b3_ref,
                 w4_ref, b4_ref, w5_ref, b5_ref, w6_ref, b6_ref, out_ref):
    wT = wt_ref[...]
    m1 = stats_ref[0:1, 0:PDIM] * (1.0 / CNT)
    m2 = stats_ref[1:1 + PDIM, 0:PDIM] * (1.0 / CNT)
    mean = (m1[:, 0:1] * wT[0:1, :] + m1[:, 1:2] * wT[1:2, :]
            + m1[:, 2:3] * wT[2:3, :])
    t = (m2[:, 0:1] * wT[0:1, :] + m2[:, 1:2] * wT[1:2, :]
         + m2[:, 2:3] * wT[2:3, :])
    ey2 = jnp.sum(wT[0:PDIM, :] * t, axis=0, keepdims=True)
    var = ey2 - mean * mean
    s = g_ref[...] * jax.lax.rsqrt(var + 1e-5)
    c = bta_ref[...] - mean * s

    pre = jnp.where(s >= 0.0, umax_ref[0] * s, umin_ref[0] * s) + c
    h = jnp.where(pre >= 0.0, pre, 0.2 * pre)
    for wref, bref in ((w1_ref, b1_ref), (w2_ref, b2_ref), (w3_ref, b3_ref),
                       (w4_ref, b4_ref), (w5_ref, b5_ref)):
        h = jax.lax.dot_general(h.astype(jnp.bfloat16), wref[...],
                                (((1,), (0,)), ((), ())),
                                preferred_element_type=jnp.float32)
        h = jnp.maximum(h + bref[...], 0.0)
    out = jax.lax.dot_general(h.astype(jnp.bfloat16), w6_ref[...],
                              (((1,), (0,)), ((), ())),
                              preferred_element_type=jnp.float32) + b6_ref[...]
    out_ref[0] = out


def kernel(x, conv1_w, bn1_gamma, bn1_beta, W1, b1, W2, b2, W3, b3,
           W4, b4, W5, b5, W6, b6):
    points = x[:, :, 0:PDIM]
    pt = jnp.transpose(points, (0, 2, 1))
    wbf = conv1_w.astype(jnp.bfloat16).astype(jnp.float32)
    wT = jnp.zeros((8, CH), jnp.float32).at[0:PDIM, :].set(wbf.T)

    umax, umin, statsb = pl.pallas_call(
        _knn_kernel,
        grid=(B, NB),
        in_specs=[
            pl.BlockSpec((1, N, PDIM), lambda b, nb: (b, 0, 0)),
            pl.BlockSpec((1, PDIM, N), lambda b, nb: (b, 0, 0)),
            pl.BlockSpec((1, RB, PDIM), lambda b, nb: (b, nb, 0)),
            pl.BlockSpec((8, CH), lambda b, nb: (0, 0)),
        ],
        out_specs=[
            pl.BlockSpec((1, RB, CH), lambda b, nb: (b, nb, 0)),
            pl.BlockSpec((1, RB, CH), lambda b, nb: (b, nb, 0)),
            pl.BlockSpec((1, 8, 128), lambda b, nb: (b, 0, 0)),
        ],
        out_shape=[
            jax.ShapeDtypeStruct((B, N, CH), jnp.float32),
            jax.ShapeDtypeStruct((B, N, CH), jnp.float32),
            jax.ShapeDtypeStruct((B, 8, 128), jnp.float32),
        ],
        compiler_params=pltpu.CompilerParams(
            dimension_semantics=("parallel", "arbitrary")),
    )(points, pt, points, wT)
    stats = jnp.sum(statsb, axis=0)

    dims = [(CH, 64), (64, 128), (128, 256), (256, 128), (128, 64), (64, 13)]
    ws = [W1.T.astype(jnp.bfloat16), W2.T.astype(jnp.bfloat16),
          W3.T.astype(jnp.bfloat16), W4.T.astype(jnp.bfloat16),
          W5.T.astype(jnp.bfloat16), W6.T.astype(jnp.bfloat16)]
    bs = [b1.reshape(1, -1), b2.reshape(1, -1), b3.reshape(1, -1),
          b4.reshape(1, -1), b5.reshape(1, -1), b6.reshape(1, -1)]

    in_specs = [
        pl.BlockSpec((1, N, CH), lambda b: (b, 0, 0)),
        pl.BlockSpec((1, N, CH), lambda b: (b, 0, 0)),
        pl.BlockSpec((8, 128), lambda b: (0, 0)),
        pl.BlockSpec((8, CH), lambda b: (0, 0)),
        pl.BlockSpec((1, CH), lambda b: (0, 0)),
        pl.BlockSpec((1, CH), lambda b: (0, 0)),
    ]
    operands = [umax, umin, stats, wT,
                bn1_gamma.reshape(1, -1), bn1_beta.reshape(1, -1)]
    for (fi, fo), w, bb in zip(dims, ws, bs):
        in_specs.append(pl.BlockSpec((fi, fo), lambda b: (0, 0)))
        in_specs.append(pl.BlockSpec((1, fo), lambda b: (0, 0)))
        operands.append(w)
        operands.append(bb)

    out = pl.pallas_call(
        _head_kernel,
        grid=(B,),
        in_specs=in_specs,
        out_specs=pl.BlockSpec((1, N, 13), lambda b: (b, 0, 0)),
        out_shape=jax.ShapeDtypeStruct((B, N, 13), jnp.float32),
    )(*operands)
    return out


# speedup vs baseline: 2.6574x; 2.6574x over previous
import jax
import jax.numpy as jnp
from jax.experimental import pallas as pl
from jax.experimental.pallas import tpu as pltpu

B, N, PDIM = 8, 2048, 3
KNN = 5
CH = 64
RB = 256
NB = N // RB
CNT = float(B * N * KNN)


def _knn_kernel(pt_ref, prow_ref, wt_ref, umax_ref, umin_ref, stats_ref):
    ptsT = pt_ref[0]
    rows = prow_ref[0]
    wT = wt_ref[...]

    sq_all = jnp.sum(ptsT * ptsT, axis=0, keepdims=True)
    sq_row = jnp.sum(rows * rows, axis=1, keepdims=True)
    rbf = rows.astype(jnp.bfloat16).astype(jnp.float32)
    tbf = ptsT.astype(jnp.bfloat16).astype(jnp.float32)
    cross = (rbf[:, 0:1] * tbf[0:1, :]
             + rbf[:, 1:2] * tbf[1:2, :]
             + rbf[:, 2:3] * tbf[2:3, :])
    d2 = sq_row + sq_all - 2.0 * cross

    iota = jax.lax.broadcasted_iota(jnp.int32, (RB, N), 1)
    umax = None
    umin = None
    m1 = jnp.zeros((1, PDIM), jnp.float32)
    m2 = jnp.zeros((PDIM, PDIM), jnp.float32)
    for r in range(KNN):
        mval = jnp.min(d2, axis=1, keepdims=True)
        idx = jnp.min(jnp.where(d2 <= mval, iota, N),
                      axis=1, keepdims=True)
        mask = iota == idx
        maskf = mask.astype(jnp.float32)
        nx = jnp.sum(maskf * ptsT[0:1, :], axis=1, keepdims=True)
        ny = jnp.sum(maskf * ptsT[1:2, :], axis=1, keepdims=True)
        nz = jnp.sum(maskf * ptsT[2:3, :], axis=1, keepdims=True)
        e = jnp.concatenate([nx, ny, nz], axis=1) - rows
        ebf = e.astype(jnp.bfloat16).astype(jnp.float32)
        z = (ebf[:, 0:1] * wT[0:1, :]
             + ebf[:, 1:2] * wT[1:2, :]
             + ebf[:, 2:3] * wT[2:3, :])
        umax = z if r == 0 else jnp.maximum(umax, z)
        umin = z if r == 0 else jnp.minimum(umin, z)
        m1 = m1 + jnp.sum(e, axis=0, keepdims=True)
        m2 = m2 + jax.lax.dot_general(e, e, (((0,), (0,)), ((), ())),
                                      preferred_element_type=jnp.float32)
        d2 = jnp.where(mask, jnp.inf, d2)

    umax_ref[0] = umax
    umin_ref[0] = umin
    blk = jnp.concatenate(
        [jnp.concatenate([m1, jnp.zeros((1, 128 - PDIM), jnp.float32)], axis=1),
         jnp.concatenate([m2, jnp.zeros((PDIM, 128 - PDIM), jnp.float32)], axis=1),
         jnp.zeros((4, 128), jnp.float32)], axis=0)
    first = pl.program_id(1) == 0

    @pl.when(first)
    def _():
        stats_ref[0] = blk

    @pl.when(jnp.logical_not(first))
    def _():
        stats_ref[0] = stats_ref[0] + blk


def _head_kernel(umax_ref, umin_ref, stats_ref, wt_ref, g_ref, bta_ref,
                 w1_ref, b1_ref, w2_ref, b2_ref, w3_ref, b3_ref,
                 w4_ref, b4_ref, w5_ref, b5_ref, w6_ref, b6_ref, out_ref):
    wT = wt_ref[...]
    m1 = stats_ref[0:1, 0:PDIM] * (1.0 / CNT)
    m2 = stats_ref[1:1 + PDIM, 0:PDIM] * (1.0 / CNT)
    mean = (m1[:, 0:1] * wT[0:1, :] + m1[:, 1:2] * wT[1:2, :]
            + m1[:, 2:3] * wT[2:3, :])
    t = (m2[:, 0:1] * wT[0:1, :] + m2[:, 1:2] * wT[1:2, :]
         + m2[:, 2:3] * wT[2:3, :])
    ey2 = jnp.sum(wT[0:PDIM, :] * t, axis=0, keepdims=True)
    var = ey2 - mean * mean
    s = g_ref[...] * jax.lax.rsqrt(var + 1e-5)
    c = bta_ref[...] - mean * s

    pre = jnp.where(s >= 0.0, umax_ref[0] * s, umin_ref[0] * s) + c
    h = jnp.where(pre >= 0.0, pre, 0.2 * pre)
    for wref, bref in ((w1_ref, b1_ref), (w2_ref, b2_ref), (w3_ref, b3_ref),
                       (w4_ref, b4_ref), (w5_ref, b5_ref)):
        h = jax.lax.dot_general(h.astype(jnp.bfloat16), wref[...],
                                (((1,), (0,)), ((), ())),
                                preferred_element_type=jnp.float32)
        h = jnp.maximum(h + bref[...], 0.0)
    out = jax.lax.dot_general(h.astype(jnp.bfloat16), w6_ref[...],
                              (((1,), (0,)), ((), ())),
                              preferred_element_type=jnp.float32) + b6_ref[...]
    out_ref[0] = out


def kernel(x, conv1_w, bn1_gamma, bn1_beta, W1, b1, W2, b2, W3, b3,
           W4, b4, W5, b5, W6, b6):
    points = x[:, :, 0:PDIM]
    pt = jnp.transpose(points, (0, 2, 1))
    wbf = conv1_w.astype(jnp.bfloat16).astype(jnp.float32)
    wT = jnp.zeros((8, CH), jnp.float32).at[0:PDIM, :].set(wbf.T)

    umax, umin, statsb = pl.pallas_call(
        _knn_kernel,
        grid=(B, NB),
        in_specs=[
            pl.BlockSpec((1, PDIM, N), lambda b, nb: (b, 0, 0)),
            pl.BlockSpec((1, RB, PDIM), lambda b, nb: (b, nb, 0)),
            pl.BlockSpec((8, CH), lambda b, nb: (0, 0)),
        ],
        out_specs=[
            pl.BlockSpec((1, RB, CH), lambda b, nb: (b, nb, 0)),
            pl.BlockSpec((1, RB, CH), lambda b, nb: (b, nb, 0)),
            pl.BlockSpec((1, 8, 128), lambda b, nb: (b, 0, 0)),
        ],
        out_shape=[
            jax.ShapeDtypeStruct((B, N, CH), jnp.float32),
            jax.ShapeDtypeStruct((B, N, CH), jnp.float32),
            jax.ShapeDtypeStruct((B, 8, 128), jnp.float32),
        ],
        compiler_params=pltpu.CompilerParams(
            dimension_semantics=("parallel", "arbitrary")),
    )(pt, points, wT)
    stats = jnp.sum(statsb, axis=0)

    dims = [(CH, 64), (64, 128), (128, 256), (256, 128), (128, 64), (64, 13)]
    ws = [W1.T.astype(jnp.bfloat16), W2.T.astype(jnp.bfloat16),
          W3.T.astype(jnp.bfloat16), W4.T.astype(jnp.bfloat16),
          W5.T.astype(jnp.bfloat16), W6.T.astype(jnp.bfloat16)]
    bs = [b1.reshape(1, -1), b2.reshape(1, -1), b3.reshape(1, -1),
          b4.reshape(1, -1), b5.reshape(1, -1), b6.reshape(1, -1)]

    in_specs = [
        pl.BlockSpec((1, N, CH), lambda b: (b, 0, 0)),
        pl.BlockSpec((1, N, CH), lambda b: (b, 0, 0)),
        pl.BlockSpec((8, 128), lambda b: (0, 0)),
        pl.BlockSpec((8, CH), lambda b: (0, 0)),
        pl.BlockSpec((1, CH), lambda b: (0, 0)),
        pl.BlockSpec((1, CH), lambda b: (0, 0)),
    ]
    operands = [umax, umin, stats, wT,
                bn1_gamma.reshape(1, -1), bn1_beta.reshape(1, -1)]
    for (fi, fo), w, bb in zip(dims, ws, bs):
        in_specs.append(pl.BlockSpec((fi, fo), lambda b: (0, 0)))
        in_specs.append(pl.BlockSpec((1, fo), lambda b: (0, 0)))
        operands.append(w)
        operands.append(bb)

    out = pl.pallas_call(
        _head_kernel,
        grid=(B,),
        in_specs=in_specs,
        out_specs=pl.BlockSpec((1, N, 13), lambda b: (b, 0, 0)),
        out_shape=jax.ShapeDtypeStruct((B, N, 13), jnp.float32),
    )(*operands)
    return out


# speedup vs baseline: 2.7988x; 1.0532x over previous
import jax
import jax.numpy as jnp
from jax.experimental import pallas as pl
from jax.experimental.pallas import tpu as pltpu

B, N, PDIM = 8, 2048, 3
KNN = 5
CH = 64
RB = 256
NB = N // RB
CNT = float(B * N * KNN)


def _knn_kernel(pt_ref, prow_ref, wt_ref, umax_ref, umin_ref, stats_ref):
    ptsT = pt_ref[0]
    rows = prow_ref[0]
    wT = wt_ref[...]

    sq_all = jnp.sum(ptsT * ptsT, axis=0, keepdims=True)
    sq_row = jnp.sum(rows * rows, axis=1, keepdims=True)
    rbf = rows.astype(jnp.bfloat16).astype(jnp.float32)
    tbf = ptsT.astype(jnp.bfloat16).astype(jnp.float32)
    cross = (rbf[:, 0:1] * tbf[0:1, :]
             + rbf[:, 1:2] * tbf[1:2, :]
             + rbf[:, 2:3] * tbf[2:3, :])
    d2 = sq_row + sq_all - 2.0 * cross

    fiota = jax.lax.broadcasted_iota(jnp.int32, (RB, N), 1).astype(jnp.float32)
    fn = jnp.float32(N)
    zero = jnp.zeros((RB, N), jnp.float32)
    umax = None
    umin = None
    m1 = jnp.zeros((1, PDIM), jnp.float32)
    m2 = jnp.zeros((PDIM, PDIM), jnp.float32)
    for r in range(KNN):
        mval = jnp.min(d2, axis=1, keepdims=True)
        fidx = jnp.min(jnp.where(d2 <= mval, fiota, fn),
                       axis=1, keepdims=True)
        mask = fiota == fidx
        nx = jnp.sum(jnp.where(mask, ptsT[0:1, :], zero), axis=1, keepdims=True)
        ny = jnp.sum(jnp.where(mask, ptsT[1:2, :], zero), axis=1, keepdims=True)
        nz = jnp.sum(jnp.where(mask, ptsT[2:3, :], zero), axis=1, keepdims=True)
        e = jnp.concatenate([nx, ny, nz], axis=1) - rows
        ebf = e.astype(jnp.bfloat16).astype(jnp.float32)
        z = (ebf[:, 0:1] * wT[0:1, :]
             + ebf[:, 1:2] * wT[1:2, :]
             + ebf[:, 2:3] * wT[2:3, :])
        umax = z if r == 0 else jnp.maximum(umax, z)
        umin = z if r == 0 else jnp.minimum(umin, z)
        m1 = m1 + jnp.sum(e, axis=0, keepdims=True)
        m2 = m2 + jax.lax.dot_general(e, e, (((0,), (0,)), ((), ())),
                                      preferred_element_type=jnp.float32)
        d2 = jnp.where(mask, jnp.inf, d2)

    umax_ref[0] = umax
    umin_ref[0] = umin
    blk = jnp.concatenate(
        [jnp.concatenate([m1, jnp.zeros((1, 128 - PDIM), jnp.float32)], axis=1),
         jnp.concatenate([m2, jnp.zeros((PDIM, 128 - PDIM), jnp.float32)], axis=1),
         jnp.zeros((4, 128), jnp.float32)], axis=0)
    first = pl.program_id(1) == 0

    @pl.when(first)
    def _():
        stats_ref[0] = blk

    @pl.when(jnp.logical_not(first))
    def _():
        stats_ref[0] = stats_ref[0] + blk


def _head_kernel(umax_ref, umin_ref, stats_ref, wt_ref, g_ref, bta_ref,
                 w1_ref, b1_ref, w2_ref, b2_ref, w3_ref, b3_ref,
                 w4_ref, b4_ref, w5_ref, b5_ref, w6_ref, b6_ref, out_ref):
    wT = wt_ref[...]
    m1 = stats_ref[0:1, 0:PDIM] * (1.0 / CNT)
    m2 = stats_ref[1:1 + PDIM, 0:PDIM] * (1.0 / CNT)
    mean = (m1[:, 0:1] * wT[0:1, :] + m1[:, 1:2] * wT[1:2, :]
            + m1[:, 2:3] * wT[2:3, :])
    t = (m2[:, 0:1] * wT[0:1, :] + m2[:, 1:2] * wT[1:2, :]
         + m2[:, 2:3] * wT[2:3, :])
    ey2 = jnp.sum(wT[0:PDIM, :] * t, axis=0, keepdims=True)
    var = ey2 - mean * mean
    s = g_ref[...] * jax.lax.rsqrt(var + 1e-5)
    c = bta_ref[...] - mean * s

    pre = jnp.where(s >= 0.0, umax_ref[0] * s, umin_ref[0] * s) + c
    h = jnp.where(pre >= 0.0, pre, 0.2 * pre)
    for wref, bref in ((w1_ref, b1_ref), (w2_ref, b2_ref), (w3_ref, b3_ref),
                       (w4_ref, b4_ref), (w5_ref, b5_ref)):
        h = jax.lax.dot_general(h.astype(jnp.bfloat16), wref[...],
                                (((1,), (0,)), ((), ())),
                                preferred_element_type=jnp.float32)
        h = jnp.maximum(h + bref[...], 0.0)
    out = jax.lax.dot_general(h.astype(jnp.bfloat16), w6_ref[...],
                              (((1,), (0,)), ((), ())),
                              preferred_element_type=jnp.float32) + b6_ref[...]
    out_ref[0] = out


def kernel(x, conv1_w, bn1_gamma, bn1_beta, W1, b1, W2, b2, W3, b3,
           W4, b4, W5, b5, W6, b6):
    points = x[:, :, 0:PDIM]
    pt = jnp.transpose(points, (0, 2, 1))
    wbf = conv1_w.astype(jnp.bfloat16).astype(jnp.float32)
    wT = jnp.zeros((8, CH), jnp.float32).at[0:PDIM, :].set(wbf.T)

    umax, umin, statsb = pl.pallas_call(
        _knn_kernel,
        grid=(B, NB),
        in_specs=[
            pl.BlockSpec((1, PDIM, N), lambda b, nb: (b, 0, 0)),
            pl.BlockSpec((1, RB, PDIM), lambda b, nb: (b, nb, 0)),
            pl.BlockSpec((8, CH), lambda b, nb: (0, 0)),
        ],
        out_specs=[
            pl.BlockSpec((1, RB, CH), lambda b, nb: (b, nb, 0)),
            pl.BlockSpec((1, RB, CH), lambda b, nb: (b, nb, 0)),
            pl.BlockSpec((1, 8, 128), lambda b, nb: (b, 0, 0)),
        ],
        out_shape=[
            jax.ShapeDtypeStruct((B, N, CH), jnp.float32),
            jax.ShapeDtypeStruct((B, N, CH), jnp.float32),
            jax.ShapeDtypeStruct((B, 8, 128), jnp.float32),
        ],
        compiler_params=pltpu.CompilerParams(
            dimension_semantics=("parallel", "arbitrary")),
    )(pt, points, wT)
    stats = jnp.sum(statsb, axis=0)

    dims = [(CH, 64), (64, 128), (128, 256), (256, 128), (128, 64), (64, 13)]
    ws = [W1.T.astype(jnp.bfloat16), W2.T.astype(jnp.bfloat16),
          W3.T.astype(jnp.bfloat16), W4.T.astype(jnp.bfloat16),
          W5.T.astype(jnp.bfloat16), W6.T.astype(jnp.bfloat16)]
    bs = [b1.reshape(1, -1), b2.reshape(1, -1), b3.reshape(1, -1),
          b4.reshape(1, -1), b5.reshape(1, -1), b6.reshape(1, -1)]

    in_specs = [
        pl.BlockSpec((1, N, CH), lambda b: (b, 0, 0)),
        pl.BlockSpec((1, N, CH), lambda b: (b, 0, 0)),
        pl.BlockSpec((8, 128), lambda b: (0, 0)),
        pl.BlockSpec((8, CH), lambda b: (0, 0)),
        pl.BlockSpec((1, CH), lambda b: (0, 0)),
        pl.BlockSpec((1, CH), lambda b: (0, 0)),
    ]
    operands = [umax, umin, stats, wT,
                bn1_gamma.reshape(1, -1), bn1_beta.reshape(1, -1)]
    for (fi, fo), w, bb in zip(dims, ws, bs):
        in_specs.append(pl.BlockSpec((fi, fo), lambda b: (0, 0)))
        in_specs.append(pl.BlockSpec((1, fo), lambda b: (0, 0)))
        operands.append(w)
        operands.append(bb)

    out = pl.pallas_call(
        _head_kernel,
        grid=(B,),
        in_specs=in_specs,
        out_specs=pl.BlockSpec((1, N, 13), lambda b: (b, 0, 0)),
        out_shape=jax.ShapeDtypeStruct((B, N, 13), jnp.float32),
    )(*operands)
    return out


# speedup vs baseline: 3.8962x; 1.3921x over previous
import functools

import jax
import jax.numpy as jnp
from jax import lax
from jax.experimental import pallas as pl
from jax.experimental.pallas import tpu as pltpu
from jax.experimental.pallas import tpu_sc as plsc

B, N, PDIM = 8, 2048, 3
KNN = 5
CH = 64
RB = 256
NB = N // RB
BN = B * N
CNT = float(BN * KNN)

GROWS = KNN * BN
NWORK = 32
WROWS = GROWS // NWORK
TROW = 128
CHUNK = 128
NCH = WROWS // CHUNK
NBUF = 4


def _knn_idx_kernel(pt_ref, prow_ref, idx_ref):
    ptsT = pt_ref[0]
    rows = prow_ref[0]
    sq_all = jnp.sum(ptsT * ptsT, axis=0, keepdims=True)
    sq_row = jnp.sum(rows * rows, axis=1, keepdims=True)
    rbf = rows.astype(jnp.bfloat16).astype(jnp.float32)
    tbf = ptsT.astype(jnp.bfloat16).astype(jnp.float32)
    cross = (rbf[:, 0:1] * tbf[0:1, :]
             + rbf[:, 1:2] * tbf[1:2, :]
             + rbf[:, 2:3] * tbf[2:3, :])
    d2 = sq_row + sq_all - 2.0 * cross

    fiota = jax.lax.broadcasted_iota(jnp.int32, (RB, N), 1).astype(jnp.float32)
    fn = jnp.float32(N)
    cols = []
    for r in range(KNN):
        mval = jnp.min(d2, axis=1, keepdims=True)
        fidx = jnp.min(jnp.where(d2 <= mval, fiota, fn),
                       axis=1, keepdims=True)
        cols.append(fidx)
        if r + 1 < KNN:
            d2 = jnp.where(fiota == fidx, jnp.inf, d2)
    base = jnp.float32(N) * pl.program_id(0).astype(jnp.float32)
    idxf = jnp.concatenate(cols, axis=1) + base
    idxf = jnp.concatenate([idxf, jnp.zeros((RB, 8 - KNN), jnp.float32)],
                           axis=1)
    idx_ref[0] = idxf.astype(jnp.int32)


def _sc_gather_body(tab_ref, idx_ref, out_ref, idx_v, rows_v, sem):
    wid = lax.axis_index("s") * 2 + lax.axis_index("c")
    pltpu.sync_copy(idx_ref.at[wid], idx_v)
    handles = [None] * NCH
    for j in range(NBUF):
        handles[j] = pltpu.async_copy(
            tab_ref.at[idx_v.at[j]], rows_v.at[j], sem.at[j])
    for j in range(NCH):
        buf = j % NBUF
        handles[j].wait()
        pltpu.sync_copy(rows_v.at[buf], out_ref.at[wid, j])
        nj = j + NBUF
        if nj < NCH:
            handles[nj] = pltpu.async_copy(
                tab_ref.at[idx_v.at[nj]], rows_v.at[buf], sem.at[buf])


_sc_gather = functools.partial(
    pl.kernel,
    mesh=plsc.VectorSubcoreMesh(core_axis_name="c", subcore_axis_name="s"),
    out_type=jax.ShapeDtypeStruct((NWORK, NCH, CHUNK, TROW), jnp.float32),
    scratch_types=[
        pltpu.VMEM((NCH, CHUNK), jnp.int32),
        pltpu.VMEM((NBUF, CHUNK, TROW), jnp.float32),
        pltpu.SemaphoreType.DMA((NBUF,)),
    ],
)(_sc_gather_body)

EB = 2048
NEB = BN // EB


def _edge_kernel(nbr_ref, p_ref, wt_ref, umax_ref, umin_ref, stats_ref):
    pts = p_ref[...]
    wT = wt_ref[...]
    umax = None
    umin = None
    m1 = jnp.zeros((1, PDIM), jnp.float32)
    m2 = jnp.zeros((PDIM, PDIM), jnp.float32)
    for k in range(KNN):
        e = nbr_ref[k][:, 0:PDIM] - pts
        ebf = e.astype(jnp.bfloat16).astype(jnp.float32)
        z = (ebf[:, 0:1] * wT[0:1, :]
             + ebf[:, 1:2] * wT[1:2, :]
             + ebf[:, 2:3] * wT[2:3, :])
        umax = z if k == 0 else jnp.maximum(umax, z)
        umin = z if k == 0 else jnp.minimum(umin, z)
        m1 = m1 + jnp.sum(e, axis=0, keepdims=True)
        m2 = m2 + jax.lax.dot_general(e, e, (((0,), (0,)), ((), ())),
                                      preferred_element_type=jnp.float32)
    umax_ref[...] = umax
    umin_ref[...] = umin
    blk = jnp.concatenate(
        [jnp.concatenate([m1, jnp.zeros((1, 128 - PDIM), jnp.float32)], axis=1),
         jnp.concatenate([m2, jnp.zeros((PDIM, 128 - PDIM), jnp.float32)], axis=1),
         jnp.zeros((4, 128), jnp.float32)], axis=0)
    first = pl.program_id(0) == 0

    @pl.when(first)
    def _():
        stats_ref[...] = blk

    @pl.when(jnp.logical_not(first))
    def _():
        stats_ref[...] = stats_ref[...] + blk


def _head_kernel(umax_ref, umin_ref, stats_ref, wt_ref, g_ref, bta_ref,
                 w1_ref, b1_ref, w2_ref, b2_ref, w3_ref, b3_ref,
                 w4_ref, b4_ref, w5_ref, b5_ref, w6_ref, b6_ref, out_ref):
    wT = wt_ref[...]
    m1 = stats_ref[0:1, 0:PDIM] * (1.0 / CNT)
    m2 = stats_ref[1:1 + PDIM, 0:PDIM] * (1.0 / CNT)
    mean = (m1[:, 0:1] * wT[0:1, :] + m1[:, 1:2] * wT[1:2, :]
            + m1[:, 2:3] * wT[2:3, :])
    t = (m2[:, 0:1] * wT[0:1, :] + m2[:, 1:2] * wT[1:2, :]
         + m2[:, 2:3] * wT[2:3, :])
    ey2 = jnp.sum(wT[0:PDIM, :] * t, axis=0, keepdims=True)
    var = ey2 - mean * mean
    s = g_ref[...] * jax.lax.rsqrt(var + 1e-5)
    c = bta_ref[...] - mean * s

    pre = jnp.where(s >= 0.0, umax_ref[0] * s, umin_ref[0] * s) + c
    h = jnp.where(pre >= 0.0, pre, 0.2 * pre)
    for wref, bref in ((w1_ref, b1_ref), (w2_ref, b2_ref), (w3_ref, b3_ref),
                       (w4_ref, b4_ref), (w5_ref, b5_ref)):
        h = jax.lax.dot_general(h.astype(jnp.bfloat16), wref[...],
                                (((1,), (0,)), ((), ())),
                                preferred_element_type=jnp.float32)
        h = jnp.maximum(h + bref[...], 0.0)
    out = jax.lax.dot_general(h.astype(jnp.bfloat16), w6_ref[...],
                              (((1,), (0,)), ((), ())),
                              preferred_element_type=jnp.float32) + b6_ref[...]
    out_ref[0] = out


def kernel(x, conv1_w, bn1_gamma, bn1_beta, W1, b1, W2, b2, W3, b3,
           W4, b4, W5, b5, W6, b6):
    points = x[:, :, 0:PDIM]
    pt = jnp.transpose(points, (0, 2, 1))
    wbf = conv1_w.astype(jnp.bfloat16).astype(jnp.float32)
    wT = jnp.zeros((8, CH), jnp.float32).at[0:PDIM, :].set(wbf.T)

    idxs = pl.pallas_call(
        _knn_idx_kernel,
        grid=(B, NB),
        in_specs=[
            pl.BlockSpec((1, PDIM, N), lambda b, nb: (b, 0, 0)),
            pl.BlockSpec((1, RB, PDIM), lambda b, nb: (b, nb, 0)),
        ],
        out_specs=pl.BlockSpec((1, RB, 8), lambda b, nb: (b, nb, 0)),
        out_shape=jax.ShapeDtypeStruct((B, N, 8), jnp.int32),
        compiler_params=pltpu.CompilerParams(
            dimension_semantics=("parallel", "arbitrary")),
    )(pt, points)

    idx3 = jnp.transpose(idxs[:, :, 0:KNN], (2, 0, 1)).reshape(
        NWORK, NCH, CHUNK)
    pflat = points.reshape(BN, PDIM)
    tab = jnp.zeros((BN, TROW), jnp.float32).at[:, 0:PDIM].set(pflat)

    nbr = _sc_gather(tab, idx3)
    nbr3 = nbr.reshape(KNN, BN, TROW)

    umax, umin, stats = pl.pallas_call(
        _edge_kernel,
        grid=(NEB,),
        in_specs=[
            pl.BlockSpec((KNN, EB, TROW), lambda i: (0, i, 0)),
            pl.BlockSpec((EB, PDIM), lambda i: (i, 0)),
            pl.BlockSpec((8, CH), lambda i: (0, 0)),
        ],
        out_specs=[
            pl.BlockSpec((EB, CH), lambda i: (i, 0)),
            pl.BlockSpec((EB, CH), lambda i: (i, 0)),
            pl.BlockSpec((8, 128), lambda i: (0, 0)),
        ],
        out_shape=[
            jax.ShapeDtypeStruct((BN, CH), jnp.float32),
            jax.ShapeDtypeStruct((BN, CH), jnp.float32),
            jax.ShapeDtypeStruct((8, 128), jnp.float32),
        ],
    )(nbr3, pflat, wT)

    dims = [(CH, 64), (64, 128), (128, 256), (256, 128), (128, 64), (64, 13)]
    ws = [W1.T.astype(jnp.bfloat16), W2.T.astype(jnp.bfloat16),
          W3.T.astype(jnp.bfloat16), W4.T.astype(jnp.bfloat16),
          W5.T.astype(jnp.bfloat16), W6.T.astype(jnp.bfloat16)]
    bs = [b1.reshape(1, -1), b2.reshape(1, -1), b3.reshape(1, -1),
          b4.reshape(1, -1), b5.reshape(1, -1), b6.reshape(1, -1)]

    in_specs = [
        pl.BlockSpec((1, N, CH), lambda b: (b, 0, 0)),
        pl.BlockSpec((1, N, CH), lambda b: (b, 0, 0)),
        pl.BlockSpec((8, 128), lambda b: (0, 0)),
        pl.BlockSpec((8, CH), lambda b: (0, 0)),
        pl.BlockSpec((1, CH), lambda b: (0, 0)),
        pl.BlockSpec((1, CH), lambda b: (0, 0)),
    ]
    operands = [umax.reshape(B, N, CH), umin.reshape(B, N, CH), stats, wT,
                bn1_gamma.reshape(1, -1), bn1_beta.reshape(1, -1)]
    for (fi, fo), w, bb in zip(dims, ws, bs):
        in_specs.append(pl.BlockSpec((fi, fo), lambda b: (0, 0)))
        in_specs.append(pl.BlockSpec((1, fo), lambda b: (0, 0)))
        operands.append(w)
        operands.append(bb)

    out = pl.pallas_call(
        _head_kernel,
        grid=(B,),
        in_specs=in_specs,
        out_specs=pl.BlockSpec((1, N, 13), lambda b: (b, 0, 0)),
        out_shape=jax.ShapeDtypeStruct((B, N, 13), jnp.float32),
    )(*operands)
    return out


# speedup vs baseline: 4.3049x; 1.1049x over previous
import functools

import jax
import jax.numpy as jnp
from jax import lax
from jax.experimental import pallas as pl
from jax.experimental.pallas import tpu as pltpu
from jax.experimental.pallas import tpu_sc as plsc

B, N, PDIM = 8, 2048, 3
KNN = 5
CH = 64
RB = 512
NB = N // RB
BN = B * N
CNT = float(BN * KNN)

GROWS = KNN * BN
NWORK = 32
WROWS = GROWS // NWORK
TROW = 128
CHUNK = 128
NCH = WROWS // CHUNK
NBUF = 4


def _knn_idx_kernel(pt_ref, prow_ref, idx_ref):
    ptsT = pt_ref[0]
    rows = prow_ref[0]
    sq_all = jnp.sum(ptsT * ptsT, axis=0, keepdims=True)
    sq_row = jnp.sum(rows * rows, axis=1, keepdims=True)
    cross = jax.lax.dot_general(
        rows.astype(jnp.bfloat16), ptsT.astype(jnp.bfloat16),
        (((1,), (0,)), ((), ())),
        preferred_element_type=jnp.float32)
    d2 = sq_row + sq_all - 2.0 * cross

    fiota = jax.lax.broadcasted_iota(jnp.int32, (RB, N), 1).astype(jnp.float32)
    fn = jnp.float32(N)
    cols = []
    for r in range(KNN):
        mval = jnp.min(d2, axis=1, keepdims=True)
        fidx = jnp.min(jnp.where(d2 <= mval, fiota, fn),
                       axis=1, keepdims=True)
        cols.append(fidx)
        if r + 1 < KNN:
            d2 = jnp.where(fiota == fidx, jnp.inf, d2)
    base = jnp.float32(N) * pl.program_id(0).astype(jnp.float32)
    idxf = jnp.concatenate(cols, axis=1) + base
    idxf = jnp.concatenate([idxf, jnp.zeros((RB, 8 - KNN), jnp.float32)],
                           axis=1)
    idx_ref[0] = idxf.astype(jnp.int32)


def _sc_gather_body(tab_ref, idx_ref, out_ref, idx_v, rows_v, sem):
    wid = lax.axis_index("s") * 2 + lax.axis_index("c")
    pltpu.sync_copy(idx_ref.at[wid], idx_v)
    handles = [None] * NCH
    for j in range(NBUF):
        handles[j] = pltpu.async_copy(
            tab_ref.at[idx_v.at[j]], rows_v.at[j], sem.at[j])
    for j in range(NCH):
        buf = j % NBUF
        handles[j].wait()
        pltpu.sync_copy(rows_v.at[buf], out_ref.at[wid, j])
        nj = j + NBUF
        if nj < NCH:
            handles[nj] = pltpu.async_copy(
                tab_ref.at[idx_v.at[nj]], rows_v.at[buf], sem.at[buf])


_sc_gather = functools.partial(
    pl.kernel,
    mesh=plsc.VectorSubcoreMesh(core_axis_name="c", subcore_axis_name="s"),
    out_type=jax.ShapeDtypeStruct((NWORK, NCH, CHUNK, TROW), jnp.float32),
    scratch_types=[
        pltpu.VMEM((NCH, CHUNK), jnp.int32),
        pltpu.VMEM((NBUF, CHUNK, TROW), jnp.float32),
        pltpu.SemaphoreType.DMA((NBUF,)),
    ],
)(_sc_gather_body)

EB = 2048
NEB = BN // EB


def _edge_kernel(nbr_ref, p_ref, wt_ref, umax_ref, umin_ref, stats_ref):
    pts = p_ref[...]
    wT = wt_ref[...]
    umax = None
    umin = None
    m1 = jnp.zeros((1, PDIM), jnp.float32)
    m2 = jnp.zeros((PDIM, PDIM), jnp.float32)
    for k in range(KNN):
        e = nbr_ref[k][:, 0:PDIM] - pts
        ebf = e.astype(jnp.bfloat16).astype(jnp.float32)
        z = (ebf[:, 0:1] * wT[0:1, :]
             + ebf[:, 1:2] * wT[1:2, :]
             + ebf[:, 2:3] * wT[2:3, :])
        umax = z if k == 0 else jnp.maximum(umax, z)
        umin = z if k == 0 else jnp.minimum(umin, z)
        m1 = m1 + jnp.sum(e, axis=0, keepdims=True)
        m2 = m2 + jax.lax.dot_general(e, e, (((0,), (0,)), ((), ())),
                                      preferred_element_type=jnp.float32)
    umax_ref[...] = umax
    umin_ref[...] = umin
    blk = jnp.concatenate(
        [jnp.concatenate([m1, jnp.zeros((1, 128 - PDIM), jnp.float32)], axis=1),
         jnp.concatenate([m2, jnp.zeros((PDIM, 128 - PDIM), jnp.float32)], axis=1),
         jnp.zeros((4, 128), jnp.float32)], axis=0)
    first = pl.program_id(0) == 0

    @pl.when(first)
    def _():
        stats_ref[...] = blk

    @pl.when(jnp.logical_not(first))
    def _():
        stats_ref[...] = stats_ref[...] + blk


def _head_kernel(umax_ref, umin_ref, stats_ref, wt_ref, g_ref, bta_ref,
                 w1_ref, b1_ref, w2_ref, b2_ref, w3_ref, b3_ref,
                 w4_ref, b4_ref, w5_ref, b5_ref, w6_ref, b6_ref, out_ref):
    wT = wt_ref[...]
    m1 = stats_ref[0:1, 0:PDIM] * (1.0 / CNT)
    m2 = stats_ref[1:1 + PDIM, 0:PDIM] * (1.0 / CNT)
    mean = (m1[:, 0:1] * wT[0:1, :] + m1[:, 1:2] * wT[1:2, :]
            + m1[:, 2:3] * wT[2:3, :])
    t = (m2[:, 0:1] * wT[0:1, :] + m2[:, 1:2] * wT[1:2, :]
         + m2[:, 2:3] * wT[2:3, :])
    ey2 = jnp.sum(wT[0:PDIM, :] * t, axis=0, keepdims=True)
    var = ey2 - mean * mean
    s = g_ref[...] * jax.lax.rsqrt(var + 1e-5)
    c = bta_ref[...] - mean * s

    pre = jnp.where(s >= 0.0, umax_ref[0] * s, umin_ref[0] * s) + c
    h = jnp.where(pre >= 0.0, pre, 0.2 * pre)
    for wref, bref in ((w1_ref, b1_ref), (w2_ref, b2_ref), (w3_ref, b3_ref),
                       (w4_ref, b4_ref), (w5_ref, b5_ref)):
        h = jax.lax.dot_general(h.astype(jnp.bfloat16), wref[...],
                                (((1,), (0,)), ((), ())),
                                preferred_element_type=jnp.float32)
        h = jnp.maximum(h + bref[...], 0.0)
    out = jax.lax.dot_general(h.astype(jnp.bfloat16), w6_ref[...],
                              (((1,), (0,)), ((), ())),
                              preferred_element_type=jnp.float32) + b6_ref[...]
    out_ref[0] = out


def kernel(x, conv1_w, bn1_gamma, bn1_beta, W1, b1, W2, b2, W3, b3,
           W4, b4, W5, b5, W6, b6):
    points = x[:, :, 0:PDIM]
    pt = jnp.transpose(points, (0, 2, 1))
    wbf = conv1_w.astype(jnp.bfloat16).astype(jnp.float32)
    wT = jnp.zeros((8, CH), jnp.float32).at[0:PDIM, :].set(wbf.T)

    idxs = pl.pallas_call(
        _knn_idx_kernel,
        grid=(B, NB),
        in_specs=[
            pl.BlockSpec((1, PDIM, N), lambda b, nb: (b, 0, 0)),
            pl.BlockSpec((1, RB, PDIM), lambda b, nb: (b, nb, 0)),
        ],
        out_specs=pl.BlockSpec((1, RB, 8), lambda b, nb: (b, nb, 0)),
        out_shape=jax.ShapeDtypeStruct((B, N, 8), jnp.int32),
        compiler_params=pltpu.CompilerParams(
            dimension_semantics=("parallel", "arbitrary")),
    )(pt, points)

    idx3 = jnp.transpose(idxs[:, :, 0:KNN], (2, 0, 1)).reshape(
        NWORK, NCH, CHUNK)
    pflat = points.reshape(BN, PDIM)
    tab = jnp.zeros((BN, TROW), jnp.float32).at[:, 0:PDIM].set(pflat)

    nbr = _sc_gather(tab, idx3)
    nbr3 = nbr.reshape(KNN, BN, TROW)

    umax, umin, stats = pl.pallas_call(
        _edge_kernel,
        grid=(NEB,),
        in_specs=[
            pl.BlockSpec((KNN, EB, TROW), lambda i: (0, i, 0)),
            pl.BlockSpec((EB, PDIM), lambda i: (i, 0)),
            pl.BlockSpec((8, CH), lambda i: (0, 0)),
        ],
        out_specs=[
            pl.BlockSpec((EB, CH), lambda i: (i, 0)),
            pl.BlockSpec((EB, CH), lambda i: (i, 0)),
            pl.BlockSpec((8, 128), lambda i: (0, 0)),
        ],
        out_shape=[
            jax.ShapeDtypeStruct((BN, CH), jnp.float32),
            jax.ShapeDtypeStruct((BN, CH), jnp.float32),
            jax.ShapeDtypeStruct((8, 128), jnp.float32),
        ],
    )(nbr3, pflat, wT)

    dims = [(CH, 64), (64, 128), (128, 256), (256, 128), (128, 64), (64, 13)]
    ws = [W1.T.astype(jnp.bfloat16), W2.T.astype(jnp.bfloat16),
          W3.T.astype(jnp.bfloat16), W4.T.astype(jnp.bfloat16),
          W5.T.astype(jnp.bfloat16), W6.T.astype(jnp.bfloat16)]
    bs = [b1.reshape(1, -1), b2.reshape(1, -1), b3.reshape(1, -1),
          b4.reshape(1, -1), b5.reshape(1, -1), b6.reshape(1, -1)]

    in_specs = [
        pl.BlockSpec((1, N, CH), lambda b: (b, 0, 0)),
        pl.BlockSpec((1, N, CH), lambda b: (b, 0, 0)),
        pl.BlockSpec((8, 128), lambda b: (0, 0)),
        pl.BlockSpec((8, CH), lambda b: (0, 0)),
        pl.BlockSpec((1, CH), lambda b: (0, 0)),
        pl.BlockSpec((1, CH), lambda b: (0, 0)),
    ]
    operands = [umax.reshape(B, N, CH), umin.reshape(B, N, CH), stats, wT,
                bn1_gamma.reshape(1, -1), bn1_beta.reshape(1, -1)]
    for (fi, fo), w, bb in zip(dims, ws, bs):
        in_specs.append(pl.BlockSpec((fi, fo), lambda b: (0, 0)))
        in_specs.append(pl.BlockSpec((1, fo), lambda b: (0, 0)))
        operands.append(w)
        operands.append(bb)

    out = pl.pallas_call(
        _head_kernel,
        grid=(B,),
        in_specs=in_specs,
        out_specs=pl.BlockSpec((1, N, 13), lambda b: (b, 0, 0)),
        out_shape=jax.ShapeDtypeStruct((B, N, 13), jnp.float32),
    )(*operands)
    return out


# speedup vs baseline: 4.6972x; 1.0911x over previous
import functools

import jax
import jax.numpy as jnp
from jax import lax
from jax.experimental import pallas as pl
from jax.experimental.pallas import tpu as pltpu
from jax.experimental.pallas import tpu_sc as plsc

B, N, PDIM = 8, 2048, 3
KNN = 5
CH = 64
RB = 512
NB = N // RB
BN = B * N
CNT = float(BN * KNN)

GROWS = KNN * BN
NWORK = 32
WROWS = GROWS // NWORK
TROW = 128
CHUNK = 128
NCH = WROWS // CHUNK
NBUF = 4


def _knn_idx_kernel(pt_ref, prow_ref, idx_ref, tab_ref):
    ptsT = pt_ref[0]
    rows = prow_ref[0]
    sq_all = jnp.sum(ptsT * ptsT, axis=0, keepdims=True)
    sq_row = jnp.sum(rows * rows, axis=1, keepdims=True)
    cross = jax.lax.dot_general(
        rows.astype(jnp.bfloat16), ptsT.astype(jnp.bfloat16),
        (((1,), (0,)), ((), ())),
        preferred_element_type=jnp.float32)
    d2 = sq_row + sq_all - 2.0 * cross

    fiota = jax.lax.broadcasted_iota(jnp.int32, (RB, N), 1).astype(jnp.float32)
    fn = jnp.float32(N)
    cols = []
    for r in range(KNN):
        mval = jnp.min(d2, axis=1, keepdims=True)
        fidx = jnp.min(jnp.where(d2 <= mval, fiota, fn),
                       axis=1, keepdims=True)
        cols.append(fidx)
        if r + 1 < KNN:
            d2 = jnp.where(fiota == fidx, jnp.inf, d2)
    base = jnp.float32(N) * pl.program_id(0).astype(jnp.float32)
    idxf = jnp.concatenate(cols, axis=1) + base
    idxf = jnp.concatenate([idxf, jnp.zeros((RB, 8 - KNN), jnp.float32)],
                           axis=1)
    idxT = jnp.transpose(idxf, (1, 0)).astype(jnp.int32)
    idx_ref[...] = idxT[0:KNN]
    tab_ref[0] = jnp.concatenate(
        [rows, jnp.zeros((RB, TROW - PDIM), jnp.float32)], axis=1)


def _sc_gather_body(tab_ref, idx_ref, out_ref, idx_v, rows_v, sem):
    wid = lax.axis_index("s") * 2 + lax.axis_index("c")
    pltpu.sync_copy(idx_ref.at[wid], idx_v)
    handles = [None] * NCH
    for j in range(NBUF):
        handles[j] = pltpu.async_copy(
            tab_ref.at[idx_v.at[j]], rows_v.at[j], sem.at[j])
    for j in range(NCH):
        buf = j % NBUF
        handles[j].wait()
        pltpu.sync_copy(rows_v.at[buf], out_ref.at[wid, j])
        nj = j + NBUF
        if nj < NCH:
            handles[nj] = pltpu.async_copy(
                tab_ref.at[idx_v.at[nj]], rows_v.at[buf], sem.at[buf])


_sc_gather = functools.partial(
    pl.kernel,
    mesh=plsc.VectorSubcoreMesh(core_axis_name="c", subcore_axis_name="s"),
    out_type=jax.ShapeDtypeStruct((NWORK, NCH, CHUNK, TROW), jnp.float32),
    scratch_types=[
        pltpu.VMEM((NCH, CHUNK), jnp.int32),
        pltpu.VMEM((NBUF, CHUNK, TROW), jnp.float32),
        pltpu.SemaphoreType.DMA((NBUF,)),
    ],
)(_sc_gather_body)

EB = 2048
NEB = BN // EB


def _edge_kernel(nbr_ref, p_ref, wt_ref, umax_ref, umin_ref, stats_ref):
    pts = p_ref[...]
    wT = wt_ref[...]
    umax = None
    umin = None
    m1 = jnp.zeros((1, PDIM), jnp.float32)
    m2 = jnp.zeros((PDIM, PDIM), jnp.float32)
    for k in range(KNN):
        e = nbr_ref[k][:, 0:PDIM] - pts
        ebf = e.astype(jnp.bfloat16).astype(jnp.float32)
        z = (ebf[:, 0:1] * wT[0:1, :]
             + ebf[:, 1:2] * wT[1:2, :]
             + ebf[:, 2:3] * wT[2:3, :])
        umax = z if k == 0 else jnp.maximum(umax, z)
        umin = z if k == 0 else jnp.minimum(umin, z)
        m1 = m1 + jnp.sum(e, axis=0, keepdims=True)
        m2 = m2 + jax.lax.dot_general(e, e, (((0,), (0,)), ((), ())),
                                      preferred_element_type=jnp.float32)
    umax_ref[...] = umax
    umin_ref[...] = umin
    blk = jnp.concatenate(
        [jnp.concatenate([m1, jnp.zeros((1, 128 - PDIM), jnp.float32)], axis=1),
         jnp.concatenate([m2, jnp.zeros((PDIM, 128 - PDIM), jnp.float32)], axis=1),
         jnp.zeros((4, 128), jnp.float32)], axis=0)
    first = pl.program_id(0) == 0

    @pl.when(first)
    def _():
        stats_ref[...] = blk

    @pl.when(jnp.logical_not(first))
    def _():
        stats_ref[...] = stats_ref[...] + blk


def _head_kernel(umax_ref, umin_ref, stats_ref, wt_ref, g_ref, bta_ref,
                 w1_ref, b1_ref, w2_ref, b2_ref, w3_ref, b3_ref,
                 w4_ref, b4_ref, w5_ref, b5_ref, w6_ref, b6_ref, out_ref):
    wT = wt_ref[...]
    m1 = stats_ref[0:1, 0:PDIM] * (1.0 / CNT)
    m2 = stats_ref[1:1 + PDIM, 0:PDIM] * (1.0 / CNT)
    mean = (m1[:, 0:1] * wT[0:1, :] + m1[:, 1:2] * wT[1:2, :]
            + m1[:, 2:3] * wT[2:3, :])
    t = (m2[:, 0:1] * wT[0:1, :] + m2[:, 1:2] * wT[1:2, :]
         + m2[:, 2:3] * wT[2:3, :])
    ey2 = jnp.sum(wT[0:PDIM, :] * t, axis=0, keepdims=True)
    var = ey2 - mean * mean
    s = g_ref[...] * jax.lax.rsqrt(var + 1e-5)
    c = bta_ref[...] - mean * s

    pre = jnp.where(s >= 0.0, umax_ref[0] * s, umin_ref[0] * s) + c
    h = jnp.where(pre >= 0.0, pre, 0.2 * pre)
    for wref, bref in ((w1_ref, b1_ref), (w2_ref, b2_ref), (w3_ref, b3_ref),
                       (w4_ref, b4_ref), (w5_ref, b5_ref)):
        h = jax.lax.dot_general(h.astype(jnp.bfloat16), wref[...],
                                (((1,), (0,)), ((), ())),
                                preferred_element_type=jnp.float32)
        h = jnp.maximum(h + bref[...], 0.0)
    out = jax.lax.dot_general(h.astype(jnp.bfloat16), w6_ref[...],
                              (((1,), (0,)), ((), ())),
                              preferred_element_type=jnp.float32) + b6_ref[...]
    out_ref[0] = out


def kernel(x, conv1_w, bn1_gamma, bn1_beta, W1, b1, W2, b2, W3, b3,
           W4, b4, W5, b5, W6, b6):
    points = x[:, :, 0:PDIM]
    pt = jnp.transpose(points, (0, 2, 1))
    wbf = conv1_w.astype(jnp.bfloat16).astype(jnp.float32)
    wT = jnp.zeros((8, CH), jnp.float32).at[0:PDIM, :].set(wbf.T)

    idxs, tab = pl.pallas_call(
        _knn_idx_kernel,
        grid=(B, NB),
        in_specs=[
            pl.BlockSpec((1, PDIM, N), lambda b, nb: (b, 0, 0)),
            pl.BlockSpec((1, RB, PDIM), lambda b, nb: (b, nb, 0)),
        ],
        out_specs=[
            pl.BlockSpec((KNN, RB), lambda b, nb: (0, b * NB + nb)),
            pl.BlockSpec((1, RB, TROW), lambda b, nb: (b, nb, 0)),
        ],
        out_shape=[
            jax.ShapeDtypeStruct((KNN, BN), jnp.int32),
            jax.ShapeDtypeStruct((B, N, TROW), jnp.float32),
        ],
        compiler_params=pltpu.CompilerParams(
            dimension_semantics=("parallel", "arbitrary")),
    )(pt, points)

    idx3 = idxs.reshape(NWORK, NCH, CHUNK)
    pflat = points.reshape(BN, PDIM)

    nbr = _sc_gather(tab.reshape(BN, TROW), idx3)
    nbr3 = nbr.reshape(KNN, BN, TROW)

    umax, umin, stats = pl.pallas_call(
        _edge_kernel,
        grid=(NEB,),
        in_specs=[
            pl.BlockSpec((KNN, EB, TROW), lambda i: (0, i, 0)),
            pl.BlockSpec((EB, PDIM), lambda i: (i, 0)),
            pl.BlockSpec((8, CH), lambda i: (0, 0)),
        ],
        out_specs=[
            pl.BlockSpec((EB, CH), lambda i: (i, 0)),
            pl.BlockSpec((EB, CH), lambda i: (i, 0)),
            pl.BlockSpec((8, 128), lambda i: (0, 0)),
        ],
        out_shape=[
            jax.ShapeDtypeStruct((BN, CH), jnp.float32),
            jax.ShapeDtypeStruct((BN, CH), jnp.float32),
            jax.ShapeDtypeStruct((8, 128), jnp.float32),
        ],
    )(nbr3, pflat, wT)

    dims = [(CH, 64), (64, 128), (128, 256), (256, 128), (128, 64), (64, 13)]
    ws = [W1.T.astype(jnp.bfloat16), W2.T.astype(jnp.bfloat16),
          W3.T.astype(jnp.bfloat16), W4.T.astype(jnp.bfloat16),
          W5.T.astype(jnp.bfloat16), W6.T.astype(jnp.bfloat16)]
    bs = [b1.reshape(1, -1), b2.reshape(1, -1), b3.reshape(1, -1),
          b4.reshape(1, -1), b5.reshape(1, -1), b6.reshape(1, -1)]

    in_specs = [
        pl.BlockSpec((1, N, CH), lambda b: (b, 0, 0)),
        pl.BlockSpec((1, N, CH), lambda b: (b, 0, 0)),
        pl.BlockSpec((8, 128), lambda b: (0, 0)),
        pl.BlockSpec((8, CH), lambda b: (0, 0)),
        pl.BlockSpec((1, CH), lambda b: (0, 0)),
        pl.BlockSpec((1, CH), lambda b: (0, 0)),
    ]
    operands = [umax.reshape(B, N, CH), umin.reshape(B, N, CH), stats, wT,
                bn1_gamma.reshape(1, -1), bn1_beta.reshape(1, -1)]
    for (fi, fo), w, bb in zip(dims, ws, bs):
        in_specs.append(pl.BlockSpec((fi, fo), lambda b: (0, 0)))
        in_specs.append(pl.BlockSpec((1, fo), lambda b: (0, 0)))
        operands.append(w)
        operands.append(bb)

    out = pl.pallas_call(
        _head_kernel,
        grid=(B,),
        in_specs=in_specs,
        out_specs=pl.BlockSpec((1, N, 13), lambda b: (b, 0, 0)),
        out_shape=jax.ShapeDtypeStruct((B, N, 13), jnp.float32),
    )(*operands)
    return out


# speedup vs baseline: 4.7823x; 1.0181x over previous
import functools

import jax
import jax.numpy as jnp
from jax import lax
from jax.experimental import pallas as pl
from jax.experimental.pallas import tpu as pltpu
from jax.experimental.pallas import tpu_sc as plsc

B, N, PDIM = 8, 2048, 3
KNN = 5
CH = 64
RB = 512
NB = N // RB
BN = B * N
CNT = float(BN * KNN)
B2 = B // 2
BN_H = B2 * N

GROWS = KNN * BN_H
NWORK = 32
WROWS = GROWS // NWORK
TROW = 128
CHUNK = 128
NCH = WROWS // CHUNK
NBUF = 4


def _knn_idx_kernel(pt_ref, prow_ref, idx_ref, tab_ref):
    ptsT = pt_ref[0]
    rows = prow_ref[0]
    sq_all = jnp.sum(ptsT * ptsT, axis=0, keepdims=True)
    sq_row = jnp.sum(rows * rows, axis=1, keepdims=True)
    cross = jax.lax.dot_general(
        rows.astype(jnp.bfloat16), ptsT.astype(jnp.bfloat16),
        (((1,), (0,)), ((), ())),
        preferred_element_type=jnp.float32)
    d2 = sq_row + sq_all - 2.0 * cross

    fiota = jax.lax.broadcasted_iota(jnp.int32, (RB, N), 1).astype(jnp.float32)
    fn = jnp.float32(N)
    cols = []
    for r in range(KNN):
        mval = jnp.min(d2, axis=1, keepdims=True)
        fidx = jnp.min(jnp.where(d2 <= mval, fiota, fn),
                       axis=1, keepdims=True)
        cols.append(fidx)
        if r + 1 < KNN:
            d2 = jnp.where(fiota == fidx, jnp.inf, d2)
    base = jnp.float32(N) * pl.program_id(0).astype(jnp.float32)
    idxf = jnp.concatenate(cols, axis=1) + base
    idxf = jnp.concatenate([idxf, jnp.zeros((RB, 8 - KNN), jnp.float32)],
                           axis=1)
    idxT = jnp.transpose(idxf, (1, 0)).astype(jnp.int32)
    idx_ref[...] = idxT[0:KNN]
    tab_ref[0] = jnp.concatenate(
        [rows, jnp.zeros((RB, TROW - PDIM), jnp.float32)], axis=1)


def _sc_gather_body(tab_ref, idx_ref, out_ref, idx_v, rows_v, sem):
    wid = lax.axis_index("s") * 2 + lax.axis_index("c")
    pltpu.sync_copy(idx_ref.at[wid], idx_v)
    handles = [None] * NCH
    for j in range(NBUF):
        handles[j] = pltpu.async_copy(
            tab_ref.at[idx_v.at[j]], rows_v.at[j], sem.at[j])
    for j in range(NCH):
        buf = j % NBUF
        handles[j].wait()
        pltpu.sync_copy(rows_v.at[buf], out_ref.at[wid, j])
        nj = j + NBUF
        if nj < NCH:
            handles[nj] = pltpu.async_copy(
                tab_ref.at[idx_v.at[nj]], rows_v.at[buf], sem.at[buf])


_sc_gather = functools.partial(
    pl.kernel,
    mesh=plsc.VectorSubcoreMesh(core_axis_name="c", subcore_axis_name="s"),
    out_type=jax.ShapeDtypeStruct((NWORK, NCH, CHUNK, TROW), jnp.float32),
    scratch_types=[
        pltpu.VMEM((NCH, CHUNK), jnp.int32),
        pltpu.VMEM((NBUF, CHUNK, TROW), jnp.float32),
        pltpu.SemaphoreType.DMA((NBUF,)),
    ],
)(_sc_gather_body)

EB = 2048
NEB = BN_H // EB


def _edge_kernel(nbr_ref, p_ref, wt_ref, umax_ref, umin_ref, stats_ref):
    pts = p_ref[...]
    wT = wt_ref[...]
    umax = None
    umin = None
    m1 = jnp.zeros((1, PDIM), jnp.float32)
    m2 = jnp.zeros((PDIM, PDIM), jnp.float32)
    for k in range(KNN):
        e = nbr_ref[k][:, 0:PDIM] - pts
        ebf = e.astype(jnp.bfloat16).astype(jnp.float32)
        z = (ebf[:, 0:1] * wT[0:1, :]
             + ebf[:, 1:2] * wT[1:2, :]
             + ebf[:, 2:3] * wT[2:3, :])
        umax = z if k == 0 else jnp.maximum(umax, z)
        umin = z if k == 0 else jnp.minimum(umin, z)
        m1 = m1 + jnp.sum(e, axis=0, keepdims=True)
        m2 = m2 + jax.lax.dot_general(e, e, (((0,), (0,)), ((), ())),
                                      preferred_element_type=jnp.float32)
    umax_ref[...] = umax
    umin_ref[...] = umin
    blk = jnp.concatenate(
        [jnp.concatenate([m1, jnp.zeros((1, 128 - PDIM), jnp.float32)], axis=1),
         jnp.concatenate([m2, jnp.zeros((PDIM, 128 - PDIM), jnp.float32)], axis=1),
         jnp.zeros((4, 128), jnp.float32)], axis=0)
    first = pl.program_id(0) == 0

    @pl.when(first)
    def _():
        stats_ref[...] = blk

    @pl.when(jnp.logical_not(first))
    def _():
        stats_ref[...] = stats_ref[...] + blk


def _head_kernel(umax_ref, umin_ref, stats_ref, stats2_ref, wt_ref, g_ref,
                 bta_ref,
                 w1_ref, b1_ref, w2_ref, b2_ref, w3_ref, b3_ref,
                 w4_ref, b4_ref, w5_ref, b5_ref, w6_ref, b6_ref, out_ref):
    wT = wt_ref[...]
    st = stats_ref[...] + stats2_ref[...]
    m1 = st[0:1, 0:PDIM] * (1.0 / CNT)
    m2 = st[1:1 + PDIM, 0:PDIM] * (1.0 / CNT)
    mean = (m1[:, 0:1] * wT[0:1, :] + m1[:, 1:2] * wT[1:2, :]
            + m1[:, 2:3] * wT[2:3, :])
    t = (m2[:, 0:1] * wT[0:1, :] + m2[:, 1:2] * wT[1:2, :]
         + m2[:, 2:3] * wT[2:3, :])
    ey2 = jnp.sum(wT[0:PDIM, :] * t, axis=0, keepdims=True)
    var = ey2 - mean * mean
    s = g_ref[...] * jax.lax.rsqrt(var + 1e-5)
    c = bta_ref[...] - mean * s

    pre = jnp.where(s >= 0.0, umax_ref[0] * s, umin_ref[0] * s) + c
    h = jnp.where(pre >= 0.0, pre, 0.2 * pre)
    for wref, bref in ((w1_ref, b1_ref), (w2_ref, b2_ref), (w3_ref, b3_ref),
                       (w4_ref, b4_ref), (w5_ref, b5_ref)):
        h = jax.lax.dot_general(h.astype(jnp.bfloat16), wref[...],
                                (((1,), (0,)), ((), ())),
                                preferred_element_type=jnp.float32)
        h = jnp.maximum(h + bref[...], 0.0)
    out = jax.lax.dot_general(h.astype(jnp.bfloat16), w6_ref[...],
                              (((1,), (0,)), ((), ())),
                              preferred_element_type=jnp.float32) + b6_ref[...]
    out_ref[0] = out


def _half_front(points_h, wT):
    pt = jnp.transpose(points_h, (0, 2, 1))
    idxs, tab = pl.pallas_call(
        _knn_idx_kernel,
        grid=(B2, NB),
        in_specs=[
            pl.BlockSpec((1, PDIM, N), lambda b, nb: (b, 0, 0)),
            pl.BlockSpec((1, RB, PDIM), lambda b, nb: (b, nb, 0)),
        ],
        out_specs=[
            pl.BlockSpec((KNN, RB), lambda b, nb: (0, b * NB + nb)),
            pl.BlockSpec((1, RB, TROW), lambda b, nb: (b, nb, 0)),
        ],
        out_shape=[
            jax.ShapeDtypeStruct((KNN, BN_H), jnp.int32),
            jax.ShapeDtypeStruct((B2, N, TROW), jnp.float32),
        ],
        compiler_params=pltpu.CompilerParams(
            dimension_semantics=("parallel", "arbitrary")),
    )(pt, points_h)

    idx3 = idxs.reshape(NWORK, NCH, CHUNK)
    pflat = points_h.reshape(BN_H, PDIM)

    nbr = _sc_gather(tab.reshape(BN_H, TROW), idx3)
    nbr3 = nbr.reshape(KNN, BN_H, TROW)

    return pl.pallas_call(
        _edge_kernel,
        grid=(NEB,),
        in_specs=[
            pl.BlockSpec((KNN, EB, TROW), lambda i: (0, i, 0)),
            pl.BlockSpec((EB, PDIM), lambda i: (i, 0)),
            pl.BlockSpec((8, CH), lambda i: (0, 0)),
        ],
        out_specs=[
            pl.BlockSpec((EB, CH), lambda i: (i, 0)),
            pl.BlockSpec((EB, CH), lambda i: (i, 0)),
            pl.BlockSpec((8, 128), lambda i: (0, 0)),
        ],
        out_shape=[
            jax.ShapeDtypeStruct((BN_H, CH), jnp.float32),
            jax.ShapeDtypeStruct((BN_H, CH), jnp.float32),
            jax.ShapeDtypeStruct((8, 128), jnp.float32),
        ],
    )(nbr3, pflat, wT)


def kernel(x, conv1_w, bn1_gamma, bn1_beta, W1, b1, W2, b2, W3, b3,
           W4, b4, W5, b5, W6, b6):
    points = x[:, :, 0:PDIM]
    wbf = conv1_w.astype(jnp.bfloat16).astype(jnp.float32)
    wT = jnp.zeros((8, CH), jnp.float32).at[0:PDIM, :].set(wbf.T)

    umaxA, uminA, statsA = _half_front(points[0:B2], wT)
    umaxB, uminB, statsB = _half_front(points[B2:B], wT)

    dims = [(CH, 64), (64, 128), (128, 256), (256, 128), (128, 64), (64, 13)]
    ws = [W1.T.astype(jnp.bfloat16), W2.T.astype(jnp.bfloat16),
          W3.T.astype(jnp.bfloat16), W4.T.astype(jnp.bfloat16),
          W5.T.astype(jnp.bfloat16), W6.T.astype(jnp.bfloat16)]
    bs = [b1.reshape(1, -1), b2.reshape(1, -1), b3.reshape(1, -1),
          b4.reshape(1, -1), b5.reshape(1, -1), b6.reshape(1, -1)]

    def head(umax, umin, sA, sB):
        in_specs = [
            pl.BlockSpec((1, N, CH), lambda b: (b, 0, 0)),
            pl.BlockSpec((1, N, CH), lambda b: (b, 0, 0)),
            pl.BlockSpec((8, 128), lambda b: (0, 0)),
            pl.BlockSpec((8, 128), lambda b: (0, 0)),
            pl.BlockSpec((8, CH), lambda b: (0, 0)),
            pl.BlockSpec((1, CH), lambda b: (0, 0)),
            pl.BlockSpec((1, CH), lambda b: (0, 0)),
        ]
        operands = [umax.reshape(B2, N, CH), umin.reshape(B2, N, CH),
                    sA, sB, wT,
                    bn1_gamma.reshape(1, -1), bn1_beta.reshape(1, -1)]
        for (fi, fo), w, bb in zip(dims, ws, bs):
            in_specs.append(pl.BlockSpec((fi, fo), lambda b: (0, 0)))
            in_specs.append(pl.BlockSpec((1, fo), lambda b: (0, 0)))
            operands.append(w)
            operands.append(bb)
        return pl.pallas_call(
            _head_kernel,
            grid=(B2,),
            in_specs=in_specs,
            out_specs=pl.BlockSpec((1, N, 13), lambda b: (b, 0, 0)),
            out_shape=jax.ShapeDtypeStruct((B2, N, 13), jnp.float32),
        )(*operands)

    outA = head(umaxA, uminA, statsA, statsB)
    outB = head(umaxB, uminB, statsA, statsB)
    return jnp.concatenate([outA, outB], axis=0)


# speedup vs baseline: 4.8660x; 1.0175x over previous
import functools

import jax
import jax.numpy as jnp
from jax import lax
from jax.experimental import pallas as pl
from jax.experimental.pallas import tpu as pltpu
from jax.experimental.pallas import tpu_sc as plsc

B, N, PDIM = 8, 2048, 3
KNN = 5
CH = 64
RB = 1024
NB = N // RB
BN = B * N
CNT = float(BN * KNN)
B2 = B // 2
BN_H = B2 * N

GROWS = KNN * BN_H
NWORK = 32
WROWS = GROWS // NWORK
TROW = 128
CHUNK = 128
NCH = WROWS // CHUNK
NBUF = 4


def _knn_idx_kernel(pt_ref, prow_ref, idx_ref, tab_ref):
    ptsT = pt_ref[0]
    rows = prow_ref[0]
    sq_all = jnp.sum(ptsT * ptsT, axis=0, keepdims=True)
    sq_row = jnp.sum(rows * rows, axis=1, keepdims=True)
    cross = jax.lax.dot_general(
        rows.astype(jnp.bfloat16), ptsT.astype(jnp.bfloat16),
        (((1,), (0,)), ((), ())),
        preferred_element_type=jnp.float32)
    d2 = sq_row + sq_all - 2.0 * cross

    fiota = jax.lax.broadcasted_iota(jnp.int32, (RB, N), 1).astype(jnp.float32)
    fn = jnp.float32(N)
    cols = []
    for r in range(KNN):
        mval = jnp.min(d2, axis=1, keepdims=True)
        fidx = jnp.min(jnp.where(d2 <= mval, fiota, fn),
                       axis=1, keepdims=True)
        cols.append(fidx)
        if r + 1 < KNN:
            d2 = jnp.where(fiota == fidx, jnp.inf, d2)
    base = jnp.float32(N) * pl.program_id(0).astype(jnp.float32)
    idxf = jnp.concatenate(cols, axis=1) + base
    idxf = jnp.concatenate([idxf, jnp.zeros((RB, 8 - KNN), jnp.float32)],
                           axis=1)
    idxT = jnp.transpose(idxf, (1, 0)).astype(jnp.int32)
    idx_ref[...] = idxT[0:KNN]
    tab_ref[0] = jnp.concatenate(
        [rows, jnp.zeros((RB, TROW - PDIM), jnp.float32)], axis=1)


def _sc_gather_body(tab_ref, idx_ref, out_ref, idx_v, rows_v, sem):
    wid = lax.axis_index("s") * 2 + lax.axis_index("c")
    pltpu.sync_copy(idx_ref.at[wid], idx_v)
    handles = [None] * NCH
    for j in range(NBUF):
        handles[j] = pltpu.async_copy(
            tab_ref.at[idx_v.at[j]], rows_v.at[j], sem.at[j])
    for j in range(NCH):
        buf = j % NBUF
        handles[j].wait()
        pltpu.sync_copy(rows_v.at[buf], out_ref.at[wid, j])
        nj = j + NBUF
        if nj < NCH:
            handles[nj] = pltpu.async_copy(
                tab_ref.at[idx_v.at[nj]], rows_v.at[buf], sem.at[buf])


_sc_gather = functools.partial(
    pl.kernel,
    mesh=plsc.VectorSubcoreMesh(core_axis_name="c", subcore_axis_name="s"),
    out_type=jax.ShapeDtypeStruct((NWORK, NCH, CHUNK, TROW), jnp.float32),
    scratch_types=[
        pltpu.VMEM((NCH, CHUNK), jnp.int32),
        pltpu.VMEM((NBUF, CHUNK, TROW), jnp.float32),
        pltpu.SemaphoreType.DMA((NBUF,)),
    ],
)(_sc_gather_body)

EB = 2048
NEB = BN_H // EB


def _edge_kernel(nbr_ref, p_ref, wt_ref, umax_ref, umin_ref, stats_ref):
    pts = p_ref[...]
    wT = wt_ref[...]
    umax = None
    umin = None
    m1 = jnp.zeros((1, PDIM), jnp.float32)
    m2 = jnp.zeros((PDIM, PDIM), jnp.float32)
    for k in range(KNN):
        e = nbr_ref[k][:, 0:PDIM] - pts
        ebf = e.astype(jnp.bfloat16).astype(jnp.float32)
        z = (ebf[:, 0:1] * wT[0:1, :]
             + ebf[:, 1:2] * wT[1:2, :]
             + ebf[:, 2:3] * wT[2:3, :])
        umax = z if k == 0 else jnp.maximum(umax, z)
        umin = z if k == 0 else jnp.minimum(umin, z)
        m1 = m1 + jnp.sum(e, axis=0, keepdims=True)
        m2 = m2 + jax.lax.dot_general(e, e, (((0,), (0,)), ((), ())),
                                      preferred_element_type=jnp.float32)
    umax_ref[...] = umax
    umin_ref[...] = umin
    blk = jnp.concatenate(
        [jnp.concatenate([m1, jnp.zeros((1, 128 - PDIM), jnp.float32)], axis=1),
         jnp.concatenate([m2, jnp.zeros((PDIM, 128 - PDIM), jnp.float32)], axis=1),
         jnp.zeros((4, 128), jnp.float32)], axis=0)
    first = pl.program_id(0) == 0

    @pl.when(first)
    def _():
        stats_ref[...] = blk

    @pl.when(jnp.logical_not(first))
    def _():
        stats_ref[...] = stats_ref[...] + blk


def _head_kernel(umax_ref, umin_ref, stats_ref, stats2_ref, wt_ref, g_ref,
                 bta_ref,
                 w1_ref, b1_ref, w2_ref, b2_ref, w3_ref, b3_ref,
                 w4_ref, b4_ref, w5_ref, b5_ref, w6_ref, b6_ref, out_ref):
    wT = wt_ref[...]
    st = stats_ref[...] + stats2_ref[...]
    m1 = st[0:1, 0:PDIM] * (1.0 / CNT)
    m2 = st[1:1 + PDIM, 0:PDIM] * (1.0 / CNT)
    mean = (m1[:, 0:1] * wT[0:1, :] + m1[:, 1:2] * wT[1:2, :]
            + m1[:, 2:3] * wT[2:3, :])
    t = (m2[:, 0:1] * wT[0:1, :] + m2[:, 1:2] * wT[1:2, :]
         + m2[:, 2:3] * wT[2:3, :])
    ey2 = jnp.sum(wT[0:PDIM, :] * t, axis=0, keepdims=True)
    var = ey2 - mean * mean
    s = g_ref[...] * jax.lax.rsqrt(var + 1e-5)
    c = bta_ref[...] - mean * s

    pre = jnp.where(s >= 0.0, umax_ref[0] * s, umin_ref[0] * s) + c
    h = jnp.where(pre >= 0.0, pre, 0.2 * pre)
    for wref, bref in ((w1_ref, b1_ref), (w2_ref, b2_ref), (w3_ref, b3_ref),
                       (w4_ref, b4_ref), (w5_ref, b5_ref)):
        h = jax.lax.dot_general(h.astype(jnp.bfloat16), wref[...],
                                (((1,), (0,)), ((), ())),
                                preferred_element_type=jnp.float32)
        h = jnp.maximum(h + bref[...], 0.0)
    out = jax.lax.dot_general(h.astype(jnp.bfloat16), w6_ref[...],
                              (((1,), (0,)), ((), ())),
                              preferred_element_type=jnp.float32) + b6_ref[...]
    out_ref[0] = out


def _half_front(points_h, wT):
    pt = jnp.transpose(points_h, (0, 2, 1))
    idxs, tab = pl.pallas_call(
        _knn_idx_kernel,
        grid=(B2, NB),
        in_specs=[
            pl.BlockSpec((1, PDIM, N), lambda b, nb: (b, 0, 0)),
            pl.BlockSpec((1, RB, PDIM), lambda b, nb: (b, nb, 0)),
        ],
        out_specs=[
            pl.BlockSpec((KNN, RB), lambda b, nb: (0, b * NB + nb)),
            pl.BlockSpec((1, RB, TROW), lambda b, nb: (b, nb, 0)),
        ],
        out_shape=[
            jax.ShapeDtypeStruct((KNN, BN_H), jnp.int32),
            jax.ShapeDtypeStruct((B2, N, TROW), jnp.float32),
        ],
        compiler_params=pltpu.CompilerParams(
            dimension_semantics=("parallel", "arbitrary")),
    )(pt, points_h)

    idx3 = idxs.reshape(NWORK, NCH, CHUNK)
    pflat = points_h.reshape(BN_H, PDIM)

    nbr = _sc_gather(tab.reshape(BN_H, TROW), idx3)
    nbr3 = nbr.reshape(KNN, BN_H, TROW)

    return pl.pallas_call(
        _edge_kernel,
        grid=(NEB,),
        in_specs=[
            pl.BlockSpec((KNN, EB, TROW), lambda i: (0, i, 0)),
            pl.BlockSpec((EB, PDIM), lambda i: (i, 0)),
            pl.BlockSpec((8, CH), lambda i: (0, 0)),
        ],
        out_specs=[
            pl.BlockSpec((EB, CH), lambda i: (i, 0)),
            pl.BlockSpec((EB, CH), lambda i: (i, 0)),
            pl.BlockSpec((8, 128), lambda i: (0, 0)),
        ],
        out_shape=[
            jax.ShapeDtypeStruct((BN_H, CH), jnp.float32),
            jax.ShapeDtypeStruct((BN_H, CH), jnp.float32),
            jax.ShapeDtypeStruct((8, 128), jnp.float32),
        ],
    )(nbr3, pflat, wT)


def kernel(x, conv1_w, bn1_gamma, bn1_beta, W1, b1, W2, b2, W3, b3,
           W4, b4, W5, b5, W6, b6):
    points = x[:, :, 0:PDIM]
    wbf = conv1_w.astype(jnp.bfloat16).astype(jnp.float32)
    wT = jnp.zeros((8, CH), jnp.float32).at[0:PDIM, :].set(wbf.T)

    umaxA, uminA, statsA = _half_front(points[0:B2], wT)
    umaxB, uminB, statsB = _half_front(points[B2:B], wT)

    dims = [(CH, 64), (64, 128), (128, 256), (256, 128), (128, 64), (64, 13)]
    ws = [W1.T.astype(jnp.bfloat16), W2.T.astype(jnp.bfloat16),
          W3.T.astype(jnp.bfloat16), W4.T.astype(jnp.bfloat16),
          W5.T.astype(jnp.bfloat16), W6.T.astype(jnp.bfloat16)]
    bs = [b1.reshape(1, -1), b2.reshape(1, -1), b3.reshape(1, -1),
          b4.reshape(1, -1), b5.reshape(1, -1), b6.reshape(1, -1)]

    def head(umax, umin, sA, sB):
        in_specs = [
            pl.BlockSpec((1, N, CH), lambda b: (b, 0, 0)),
            pl.BlockSpec((1, N, CH), lambda b: (b, 0, 0)),
            pl.BlockSpec((8, 128), lambda b: (0, 0)),
            pl.BlockSpec((8, 128), lambda b: (0, 0)),
            pl.BlockSpec((8, CH), lambda b: (0, 0)),
            pl.BlockSpec((1, CH), lambda b: (0, 0)),
            pl.BlockSpec((1, CH), lambda b: (0, 0)),
        ]
        operands = [umax.reshape(B2, N, CH), umin.reshape(B2, N, CH),
                    sA, sB, wT,
                    bn1_gamma.reshape(1, -1), bn1_beta.reshape(1, -1)]
        for (fi, fo), w, bb in zip(dims, ws, bs):
            in_specs.append(pl.BlockSpec((fi, fo), lambda b: (0, 0)))
            in_specs.append(pl.BlockSpec((1, fo), lambda b: (0, 0)))
            operands.append(w)
            operands.append(bb)
        return pl.pallas_call(
            _head_kernel,
            grid=(B2,),
            in_specs=in_specs,
            out_specs=pl.BlockSpec((1, N, 13), lambda b: (b, 0, 0)),
            out_shape=jax.ShapeDtypeStruct((B2, N, 13), jnp.float32),
        )(*operands)

    outA = head(umaxA, uminA, statsA, statsB)
    outB = head(umaxB, uminB, statsA, statsB)
    return jnp.concatenate([outA, outB], axis=0)
